# baseline (device time: 4401747 ns/iter reference)
import jax
import jax.numpy as jnp
from jax import lax
from jax.experimental import pallas as pl
from jax.experimental.pallas import tpu as pltpu

N_DEV = 4


def _ring_allgather(x_shard):
    m_per, k = x_shard.shape

    def body(x_ref, out_ref, send_sems, recv_sems, copy_sem):
        my_pos = lax.axis_index("i")
        left = lax.rem(my_pos + N_DEV - 1, N_DEV)
        right = lax.rem(my_pos + 1, N_DEV)

        barrier_sem = pltpu.get_barrier_semaphore()
        for nbr in (left, right):
            pl.semaphore_signal(
                barrier_sem, inc=1,
                device_id=(nbr,), device_id_type=pl.DeviceIdType.MESH,
            )
        pl.semaphore_wait(barrier_sem, 2)

        local = pltpu.make_async_copy(
            x_ref, out_ref.at[pl.ds(my_pos * m_per, m_per), :], copy_sem
        )
        local.start()
        local.wait()

        for h in range(N_DEV - 1):
            origin = lax.rem(my_pos + N_DEV - h, N_DEV)
            sl = pl.ds(origin * m_per, m_per)
            rdma = pltpu.make_async_remote_copy(
                src_ref=out_ref.at[sl, :],
                dst_ref=out_ref.at[sl, :],
                send_sem=send_sems.at[h],
                recv_sem=recv_sems.at[h],
                device_id=(right,),
                device_id_type=pl.DeviceIdType.MESH,
            )
            rdma.start()
            rdma.wait()

    return pl.pallas_call(
        body,
        out_shape=jax.ShapeDtypeStruct((N_DEV * m_per, k), x_shard.dtype),
        in_specs=[pl.BlockSpec(memory_space=pltpu.MemorySpace.HBM)],
        out_specs=pl.BlockSpec(memory_space=pltpu.MemorySpace.HBM),
        scratch_shapes=[
            pltpu.SemaphoreType.DMA((N_DEV - 1,)),
            pltpu.SemaphoreType.DMA((N_DEV - 1,)),
            pltpu.SemaphoreType.DMA,
        ],
        compiler_params=pltpu.CompilerParams(collective_id=0),
    )(x_shard)


def kernel(x, w_mat):
    x_full = _ring_allgather(x)
    y = jnp.dot(x_full, w_mat, preferred_element_type=jnp.float32)
    return jnp.maximum(y, 0.0)


# device time: 691473 ns/iter; 6.3658x vs baseline; 6.3658x over previous
import jax
import jax.numpy as jnp
from jax import lax
from jax.experimental import pallas as pl
from jax.experimental.pallas import tpu as pltpu

N_DEV = 4
TILE = 256


def kernel(x, w_mat):
    x = x.astype(jnp.bfloat16)
    w_mat = w_mat.astype(jnp.bfloat16)
    m_per, k = x.shape
    n = w_mat.shape[1]
    half = m_per // 2
    m_out = N_DEV * m_per

    def body(x_ref, w_ref, out_ref,
             cw_chunk, cw_half, ccw_chunk, ccw_half,
             x_tiles, y_tile,
             send_sems, recv_sems, load_sems, store_sem):
        my = lax.axis_index("i")
        left = lax.rem(my + N_DEV - 1, N_DEV)
        right = lax.rem(my + 1, N_DEV)
        opp = lax.rem(my + 2, N_DEV)

        bar = pltpu.get_barrier_semaphore()
        for nbr in (left, right):
            pl.semaphore_signal(bar, inc=1, device_id=(nbr,),
                                device_id_type=pl.DeviceIdType.MESH)
        pl.semaphore_wait(bar, 2)

        send_cw = pltpu.make_async_remote_copy(
            src_ref=x_ref, dst_ref=cw_chunk,
            send_sem=send_sems.at[0], recv_sem=recv_sems.at[0],
            device_id=(right,), device_id_type=pl.DeviceIdType.MESH)
        send_ccw = pltpu.make_async_remote_copy(
            src_ref=x_ref, dst_ref=ccw_chunk,
            send_sem=send_sems.at[1], recv_sem=recv_sems.at[1],
            device_id=(left,), device_id_type=pl.DeviceIdType.MESH)
        send_cw.start()
        send_ccw.start()

        def run_tiles(src_ref, r0, o0, nt):
            def step(t, carry):
                ld = pltpu.make_async_copy(
                    src_ref.at[pl.ds(r0 + t * TILE, TILE), :],
                    x_tiles.at[0], load_sems.at[0])
                ld.start()
                ld.wait()
                y = jnp.dot(x_tiles[0], w_ref[...],
                            preferred_element_type=jnp.float32)
                y_tile[...] = jnp.maximum(y, 0.0)
                st = pltpu.make_async_copy(
                    y_tile, out_ref.at[pl.ds(o0 + t * TILE, TILE), :],
                    store_sem)
                st.start()
                st.wait()
                return carry
            lax.fori_loop(0, nt, step, 0)

        run_tiles(x_ref, 0, my * m_per, m_per // TILE)

        send_cw.wait_recv()
        fwd_cw = pltpu.make_async_remote_copy(
            src_ref=cw_chunk.at[pl.ds(0, half), :], dst_ref=cw_half,
            send_sem=send_sems.at[2], recv_sem=recv_sems.at[2],
            device_id=(right,), device_id_type=pl.DeviceIdType.MESH)
        fwd_cw.start()
        send_ccw.wait_recv()
        fwd_ccw = pltpu.make_async_remote_copy(
            src_ref=ccw_chunk.at[pl.ds(half, half), :], dst_ref=ccw_half,
            send_sem=send_sems.at[3], recv_sem=recv_sems.at[3],
            device_id=(left,), device_id_type=pl.DeviceIdType.MESH)
        fwd_ccw.start()

        run_tiles(cw_chunk, 0, left * m_per, m_per // TILE)
        run_tiles(ccw_chunk, 0, right * m_per, m_per // TILE)

        fwd_cw.wait_recv()
        run_tiles(cw_half, 0, opp * m_per, half // TILE)
        fwd_ccw.wait_recv()
        run_tiles(ccw_half, 0, opp * m_per + half, half // TILE)

        send_cw.wait_send()
        send_ccw.wait_send()
        fwd_cw.wait_send()
        fwd_ccw.wait_send()

    hbm = pltpu.MemorySpace.HBM
    out = pl.pallas_call(
        body,
        out_shape=[
            jax.ShapeDtypeStruct((m_out, n), jnp.float32),
            jax.ShapeDtypeStruct((m_per, k), jnp.bfloat16),
            jax.ShapeDtypeStruct((half, k), jnp.bfloat16),
            jax.ShapeDtypeStruct((m_per, k), jnp.bfloat16),
            jax.ShapeDtypeStruct((half, k), jnp.bfloat16),
        ],
        in_specs=[pl.BlockSpec(memory_space=hbm),
                  pl.BlockSpec(memory_space=pltpu.MemorySpace.VMEM)],
        out_specs=[pl.BlockSpec(memory_space=hbm)] * 5,
        scratch_shapes=[
            pltpu.VMEM((1, TILE, k), jnp.bfloat16),
            pltpu.VMEM((TILE, n), jnp.float32),
            pltpu.SemaphoreType.DMA((4,)),
            pltpu.SemaphoreType.DMA((4,)),
            pltpu.SemaphoreType.DMA((1,)),
            pltpu.SemaphoreType.DMA,
        ],
        compiler_params=pltpu.CompilerParams(
            collective_id=0,
            vmem_limit_bytes=50 * 1024 * 1024,
        ),
    )(x, w_mat)
    return out[0]


# device time: 677931 ns/iter; 6.4929x vs baseline; 1.0200x over previous
import jax
import jax.numpy as jnp
from jax import lax
from jax.experimental import pallas as pl
from jax.experimental.pallas import tpu as pltpu

N_DEV = 4
TILE = 256


def kernel(x, w_mat):
    x = x.astype(jnp.bfloat16)
    w_mat = w_mat.astype(jnp.bfloat16)
    m_per, k = x.shape
    n = w_mat.shape[1]
    half = m_per // 2
    m_out = N_DEV * m_per

    def body(x_ref, w_ref, out_ref,
             cw_chunk, cw_half, ccw_chunk, ccw_half,
             x_tiles, y_tile,
             send_sems, recv_sems, load_sems, store_sem):
        my = lax.axis_index("i")
        left = lax.rem(my + N_DEV - 1, N_DEV)
        right = lax.rem(my + 1, N_DEV)
        opp = lax.rem(my + 2, N_DEV)

        bar = pltpu.get_barrier_semaphore()
        for nbr in (left, right):
            pl.semaphore_signal(bar, inc=1, device_id=(nbr,),
                                device_id_type=pl.DeviceIdType.MESH)
        pl.semaphore_wait(bar, 2)

        send_cw = pltpu.make_async_remote_copy(
            src_ref=x_ref, dst_ref=cw_chunk,
            send_sem=send_sems.at[0], recv_sem=recv_sems.at[0],
            device_id=(right,), device_id_type=pl.DeviceIdType.MESH)
        send_ccw = pltpu.make_async_remote_copy(
            src_ref=x_ref, dst_ref=ccw_chunk,
            send_sem=send_sems.at[1], recv_sem=recv_sems.at[1],
            device_id=(left,), device_id_type=pl.DeviceIdType.MESH)
        send_cw.start()
        send_ccw.start()

        def run_tiles(src_ref, r0, o0, nt):
            assert nt % 2 == 0

            def load(t, slot):
                return pltpu.make_async_copy(
                    src_ref.at[pl.ds(r0 + t * TILE, TILE), :],
                    x_tiles.at[slot], load_sems.at[slot])

            def compute(slot, t):
                y = jnp.dot(x_tiles[slot], w_ref[...],
                            preferred_element_type=jnp.float32)
                y_tile[...] = jnp.maximum(y, 0.0)
                st = pltpu.make_async_copy(
                    y_tile, out_ref.at[pl.ds(o0 + t * TILE, TILE), :],
                    store_sem)
                st.start()
                st.wait()

            load(0, 0).start()

            def step(i, carry):
                t0 = 2 * i
                load(t0, 0).wait()
                load(t0 + 1, 1).start()
                compute(0, t0)
                load(t0 + 1, 1).wait()

                @pl.when(t0 + 2 < nt)
                def _():
                    load(t0 + 2, 0).start()

                compute(1, t0 + 1)
                return carry

            lax.fori_loop(0, nt // 2, step, 0)

        run_tiles(x_ref, 0, my * m_per, m_per // TILE)

        send_cw.wait_recv()
        fwd_cw = pltpu.make_async_remote_copy(
            src_ref=cw_chunk.at[pl.ds(0, half), :], dst_ref=cw_half,
            send_sem=send_sems.at[2], recv_sem=recv_sems.at[2],
            device_id=(right,), device_id_type=pl.DeviceIdType.MESH)
        fwd_cw.start()
        send_ccw.wait_recv()
        fwd_ccw = pltpu.make_async_remote_copy(
            src_ref=ccw_chunk.at[pl.ds(half, half), :], dst_ref=ccw_half,
            send_sem=send_sems.at[3], recv_sem=recv_sems.at[3],
            device_id=(left,), device_id_type=pl.DeviceIdType.MESH)
        fwd_ccw.start()

        run_tiles(cw_chunk, 0, left * m_per, m_per // TILE)
        run_tiles(ccw_chunk, 0, right * m_per, m_per // TILE)

        fwd_cw.wait_recv()
        run_tiles(cw_half, 0, opp * m_per, half // TILE)
        fwd_ccw.wait_recv()
        run_tiles(ccw_half, 0, opp * m_per + half, half // TILE)

        send_cw.wait_send()
        send_ccw.wait_send()
        fwd_cw.wait_send()
        fwd_ccw.wait_send()

    hbm = pltpu.MemorySpace.HBM
    out = pl.pallas_call(
        body,
        out_shape=[
            jax.ShapeDtypeStruct((m_out, n), jnp.float32),
            jax.ShapeDtypeStruct((m_per, k), jnp.bfloat16),
            jax.ShapeDtypeStruct((half, k), jnp.bfloat16),
            jax.ShapeDtypeStruct((m_per, k), jnp.bfloat16),
            jax.ShapeDtypeStruct((half, k), jnp.bfloat16),
        ],
        in_specs=[pl.BlockSpec(memory_space=hbm),
                  pl.BlockSpec(memory_space=pltpu.MemorySpace.VMEM)],
        out_specs=[pl.BlockSpec(memory_space=hbm)] * 5,
        scratch_shapes=[
            pltpu.VMEM((2, TILE, k), jnp.bfloat16),
            pltpu.VMEM((TILE, n), jnp.float32),
            pltpu.SemaphoreType.DMA((4,)),
            pltpu.SemaphoreType.DMA((4,)),
            pltpu.SemaphoreType.DMA((2,)),
            pltpu.SemaphoreType.DMA,
        ],
        compiler_params=pltpu.CompilerParams(
            collective_id=0,
            vmem_limit_bytes=50 * 1024 * 1024,
        ),
    )(x, w_mat)
    return out[0]


# device time: 653044 ns/iter; 6.7404x vs baseline; 1.0381x over previous
import jax
import jax.numpy as jnp
from jax import lax
from jax.experimental import pallas as pl
from jax.experimental.pallas import tpu as pltpu

N_DEV = 4
TILE = 256
CAST = 128
QTR = 256


def kernel(x, w_mat):
    w_mat = w_mat.astype(jnp.bfloat16)
    m_per, k = x.shape
    n = w_mat.shape[1]
    half = m_per // 2
    m_out = N_DEV * m_per

    def body(x_ref, w_ref, out_ref, x_full,
             xf32_tiles, xbf_tiles, x_tile, y_tile,
             send_sems, recv_sems, cast_sems, stage_sems,
             load_sem, store_sem):
        my = lax.axis_index("i")
        left = lax.rem(my + N_DEV - 1, N_DEV)
        right = lax.rem(my + 1, N_DEV)
        opp = lax.rem(my + 2, N_DEV)
        my0 = my * m_per
        left0 = left * m_per
        right0 = right * m_per
        opp0 = opp * m_per

        bar = pltpu.get_barrier_semaphore()
        for nbr in (left, right):
            pl.semaphore_signal(bar, inc=1, device_id=(nbr,),
                                device_id_type=pl.DeviceIdType.MESH)
        pl.semaphore_wait(bar, 2)

        def cast_rows(t0_rows, nt):
            def ld(t, slot):
                return pltpu.make_async_copy(
                    x_ref.at[pl.ds(t0_rows + t * CAST, CAST), :],
                    xf32_tiles.at[slot], cast_sems.at[slot])

            def cast_store(t, slot):
                xbf_tiles[slot] = xf32_tiles[slot].astype(jnp.bfloat16)
                st = pltpu.make_async_copy(
                    xbf_tiles.at[slot],
                    x_full.at[pl.ds(my0 + t0_rows + t * CAST, CAST), :],
                    stage_sems.at[slot])
                st.start()
                st.wait()

            ld(0, 0).start()

            def step(i, carry):
                t0 = 2 * i
                ld(t0, 0).wait()
                ld(t0 + 1, 1).start()
                cast_store(t0, 0)
                ld(t0 + 1, 1).wait()

                @pl.when(t0 + 2 < nt)
                def _():
                    ld(t0 + 2, 0).start()

                cast_store(t0 + 1, 1)
                return carry

            lax.fori_loop(0, nt // 2, step, 0)

        def rdma(rows0, nrows, dst, recv_slot, send_slot):
            return pltpu.make_async_remote_copy(
                src_ref=x_full.at[pl.ds(rows0, nrows), :],
                dst_ref=x_full.at[pl.ds(rows0, nrows), :],
                send_sem=send_sems.at[send_slot],
                recv_sem=recv_sems.at[recv_slot],
                device_id=(dst,), device_id_type=pl.DeviceIdType.MESH)

        s_own_a = rdma(my0, half, right, 0, 0)
        s_own_b = rdma(my0 + half, half, left, 1, 1)
        s_fwd_a = rdma(left0, half, right, 2, 2)
        s_fwd_b = rdma(right0 + half, half, left, 3, 3)
        s_qtr_b = [rdma(my0 + half + q * QTR, QTR, right, 4 + q, 4 + q)
                   for q in range(4)]
        s_qtr_a = [rdma(my0 + q * QTR, QTR, left, 8 + q, 8 + q)
                   for q in range(4)]
        r_left_a = rdma(left0, half, left, 0, 0)
        r_right_b = rdma(right0 + half, half, right, 1, 1)
        r_opp_a = rdma(opp0, half, left, 2, 2)
        r_opp_b = rdma(opp0 + half, half, right, 3, 3)
        r_qtr_b = [rdma(left0 + half + q * QTR, QTR, left, 4 + q, 4 + q)
                   for q in range(4)]
        r_qtr_a = [rdma(right0 + q * QTR, QTR, right, 8 + q, 8 + q)
                   for q in range(4)]

        cast_rows(0, half // CAST)
        s_own_a.start()
        cast_rows(half, half // CAST)
        s_own_b.start()

        def tile_index(j):
            mt, lt, rt, ot = (my0 // TILE, left0 // TILE,
                              right0 // TILE, opp0 // TILE)
            ht = half // TILE
            q2 = (j - 24) // 2
            return jnp.where(
                j < 8, mt + j,
                jnp.where(
                    j < 12, lt + (j - 8),
                    jnp.where(
                        j < 16, rt + ht + (j - 12),
                        jnp.where(
                            j < 20, ot + (j - 16),
                            jnp.where(
                                j < 24, ot + ht + (j - 20),
                                jnp.where(
                                    lax.rem(j, 2) == 0,
                                    lt + ht + q2,
                                    rt + q2))))))

        def gemm_step(j, carry):
            @pl.when(j == 8)
            def _():
                r_left_a.wait_recv()
                s_fwd_a.start()
                for s in s_qtr_b:
                    s.start()
                r_right_b.wait_recv()
                s_fwd_b.start()
                for s in s_qtr_a:
                    s.start()

            @pl.when(j == 16)
            def _():
                r_opp_a.wait_recv()

            @pl.when(j == 20)
            def _():
                r_opp_b.wait_recv()

            for jq in range(24, 32):
                @pl.when(j == jq)
                def _(jq=jq):
                    if jq % 2 == 0:
                        r_qtr_b[(jq - 24) // 2].wait_recv()
                    else:
                        r_qtr_a[(jq - 24) // 2].wait_recv()

            r0 = tile_index(j) * TILE
            ld = pltpu.make_async_copy(
                x_full.at[pl.ds(r0, TILE), :], x_tile, load_sem)
            ld.start()
            ld.wait()
            y = jnp.dot(x_tile[...], w_ref[...],
                        preferred_element_type=jnp.float32)
            y_tile[...] = jnp.maximum(y, 0.0)
            st = pltpu.make_async_copy(
                y_tile, out_ref.at[pl.ds(r0, TILE), :], store_sem)
            st.start()
            st.wait()
            return carry

        lax.fori_loop(0, (m_out // TILE), gemm_step, 0)

        for s in [s_own_a, s_own_b, s_fwd_a, s_fwd_b] + s_qtr_b + s_qtr_a:
            s.wait_send()

    hbm = pltpu.MemorySpace.HBM
    out = pl.pallas_call(
        body,
        out_shape=[
            jax.ShapeDtypeStruct((m_out, n), jnp.float32),
            jax.ShapeDtypeStruct((m_out, k), jnp.bfloat16),
        ],
        in_specs=[pl.BlockSpec(memory_space=hbm),
                  pl.BlockSpec(memory_space=pltpu.MemorySpace.VMEM)],
        out_specs=[pl.BlockSpec(memory_space=hbm)] * 2,
        scratch_shapes=[
            pltpu.VMEM((2, CAST, k), jnp.float32),
            pltpu.VMEM((2, CAST, k), jnp.bfloat16),
            pltpu.VMEM((TILE, k), jnp.bfloat16),
            pltpu.VMEM((TILE, n), jnp.float32),
            pltpu.SemaphoreType.DMA((12,)),
            pltpu.SemaphoreType.DMA((12,)),
            pltpu.SemaphoreType.DMA((2,)),
            pltpu.SemaphoreType.DMA((2,)),
            pltpu.SemaphoreType.DMA,
            pltpu.SemaphoreType.DMA,
        ],
        compiler_params=pltpu.CompilerParams(
            collective_id=0,
            vmem_limit_bytes=56 * 1024 * 1024,
        ),
    )(x, w_mat)
    return out[0]


# device time: 653029 ns/iter; 6.7405x vs baseline; 1.0000x over previous
import jax
import jax.numpy as jnp
from jax import lax
from jax.experimental import pallas as pl
from jax.experimental.pallas import tpu as pltpu

N_DEV = 4
TILE = 256
CAST = 128
QTR = 256


def kernel(x, w_mat):
    w_mat = w_mat.astype(jnp.bfloat16)
    m_per, k = x.shape
    n = w_mat.shape[1]
    half = m_per // 2
    m_out = N_DEV * m_per

    def body(x_ref, w_ref, out_ref, x_full,
             xf32_tiles, xbf_tiles, x_tile, y_tile,
             send_sems, recv_sems, cast_sems, stage_sems,
             load_sem, store_sem):
        my = lax.axis_index("i")
        left = lax.rem(my + N_DEV - 1, N_DEV)
        right = lax.rem(my + 1, N_DEV)
        opp = lax.rem(my + 2, N_DEV)
        my0 = my * m_per
        left0 = left * m_per
        right0 = right * m_per
        opp0 = opp * m_per

        bar = pltpu.get_barrier_semaphore()
        for nbr in (left, right):
            pl.semaphore_signal(bar, inc=1, device_id=(nbr,),
                                device_id_type=pl.DeviceIdType.MESH)
        pl.semaphore_wait(bar, 2)

        def cast_rows(t0_rows, nt):
            def ld(t, slot):
                return pltpu.make_async_copy(
                    x_ref.at[pl.ds(t0_rows + t * CAST, CAST), :],
                    xf32_tiles.at[slot], cast_sems.at[slot])

            def st_desc(t, slot):
                return pltpu.make_async_copy(
                    xbf_tiles.at[slot],
                    x_full.at[pl.ds(my0 + t0_rows + t * CAST, CAST), :],
                    stage_sems.at[slot])

            ld(0, 0).start()

            def step(i, carry):
                t0 = 2 * i
                ld(t0, 0).wait()
                ld(t0 + 1, 1).start()

                @pl.when(t0 >= 2)
                def _():
                    st_desc(t0 - 2, 0).wait()

                xbf_tiles[0] = xf32_tiles[0].astype(jnp.bfloat16)
                st_desc(t0, 0).start()
                ld(t0 + 1, 1).wait()

                @pl.when(t0 + 2 < nt)
                def _():
                    ld(t0 + 2, 0).start()

                @pl.when(t0 >= 2)
                def _():
                    st_desc(t0 - 1, 1).wait()

                xbf_tiles[1] = xf32_tiles[1].astype(jnp.bfloat16)
                st_desc(t0 + 1, 1).start()
                return carry

            lax.fori_loop(0, nt // 2, step, 0)
            st_desc(nt - 2, 0).wait()
            st_desc(nt - 1, 1).wait()

        def rdma(rows0, nrows, dst, recv_slot, send_slot):
            return pltpu.make_async_remote_copy(
                src_ref=x_full.at[pl.ds(rows0, nrows), :],
                dst_ref=x_full.at[pl.ds(rows0, nrows), :],
                send_sem=send_sems.at[send_slot],
                recv_sem=recv_sems.at[recv_slot],
                device_id=(dst,), device_id_type=pl.DeviceIdType.MESH)

        s_own_a = rdma(my0, half, right, 0, 0)
        s_own_b = rdma(my0 + half, half, left, 1, 1)
        s_fwd_a = rdma(left0, half, right, 2, 2)
        s_fwd_b = rdma(right0 + half, half, left, 3, 3)
        s_qtr_b = [rdma(my0 + half + q * QTR, QTR, right, 4 + q, 4 + q)
                   for q in range(4)]
        s_qtr_a = [rdma(my0 + q * QTR, QTR, left, 8 + q, 8 + q)
                   for q in range(4)]
        r_left_a = rdma(left0, half, left, 0, 0)
        r_right_b = rdma(right0 + half, half, right, 1, 1)
        r_opp_a = rdma(opp0, half, left, 2, 2)
        r_opp_b = rdma(opp0 + half, half, right, 3, 3)
        r_qtr_b = [rdma(left0 + half + q * QTR, QTR, left, 4 + q, 4 + q)
                   for q in range(4)]
        r_qtr_a = [rdma(right0 + q * QTR, QTR, right, 8 + q, 8 + q)
                   for q in range(4)]

        cast_rows(0, half // CAST)
        s_own_a.start()
        cast_rows(half, half // CAST)
        s_own_b.start()

        def tile_index(j):
            mt, lt, rt, ot = (my0 // TILE, left0 // TILE,
                              right0 // TILE, opp0 // TILE)
            ht = half // TILE
            q2 = (j - 24) // 2
            return jnp.where(
                j < 8, mt + j,
                jnp.where(
                    j < 12, lt + (j - 8),
                    jnp.where(
                        j < 16, rt + ht + (j - 12),
                        jnp.where(
                            j < 20, ot + (j - 16),
                            jnp.where(
                                j < 24, ot + ht + (j - 20),
                                jnp.where(
                                    lax.rem(j, 2) == 0,
                                    lt + ht + q2,
                                    rt + q2))))))

        def gemm_step(j, carry):
            @pl.when(j == 8)
            def _():
                r_left_a.wait_recv()
                s_fwd_a.start()
                for s in s_qtr_b:
                    s.start()
                r_right_b.wait_recv()
                s_fwd_b.start()
                for s in s_qtr_a:
                    s.start()

            @pl.when(j == 16)
            def _():
                r_opp_a.wait_recv()

            @pl.when(j == 20)
            def _():
                r_opp_b.wait_recv()

            for jq in range(24, 32):
                @pl.when(j == jq)
                def _(jq=jq):
                    if jq % 2 == 0:
                        r_qtr_b[(jq - 24) // 2].wait_recv()
                    else:
                        r_qtr_a[(jq - 24) // 2].wait_recv()

            r0 = tile_index(j) * TILE
            ld = pltpu.make_async_copy(
                x_full.at[pl.ds(r0, TILE), :], x_tile, load_sem)
            ld.start()
            ld.wait()
            y = jnp.dot(x_tile[...], w_ref[...],
                        preferred_element_type=jnp.float32)
            y_tile[...] = jnp.maximum(y, 0.0)
            st = pltpu.make_async_copy(
                y_tile, out_ref.at[pl.ds(r0, TILE), :], store_sem)
            st.start()
            st.wait()
            return carry

        lax.fori_loop(0, (m_out // TILE), gemm_step, 0)

        for s in [s_own_a, s_own_b, s_fwd_a, s_fwd_b] + s_qtr_b + s_qtr_a:
            s.wait_send()

    hbm = pltpu.MemorySpace.HBM
    out = pl.pallas_call(
        body,
        out_shape=[
            jax.ShapeDtypeStruct((m_out, n), jnp.float32),
            jax.ShapeDtypeStruct((m_out, k), jnp.bfloat16),
        ],
        in_specs=[pl.BlockSpec(memory_space=hbm),
                  pl.BlockSpec(memory_space=pltpu.MemorySpace.VMEM)],
        out_specs=[pl.BlockSpec(memory_space=hbm)] * 2,
        scratch_shapes=[
            pltpu.VMEM((2, CAST, k), jnp.float32),
            pltpu.VMEM((2, CAST, k), jnp.bfloat16),
            pltpu.VMEM((TILE, k), jnp.bfloat16),
            pltpu.VMEM((TILE, n), jnp.float32),
            pltpu.SemaphoreType.DMA((12,)),
            pltpu.SemaphoreType.DMA((12,)),
            pltpu.SemaphoreType.DMA((2,)),
            pltpu.SemaphoreType.DMA((2,)),
            pltpu.SemaphoreType.DMA,
            pltpu.SemaphoreType.DMA,
        ],
        compiler_params=pltpu.CompilerParams(
            collective_id=0,
            vmem_limit_bytes=56 * 1024 * 1024,
        ),
    )(x, w_mat)
    return out[0]
